# baseline (speedup 1.0000x reference)
import functools

import jax
import jax.numpy as jnp
from jax import lax
from jax.experimental import pallas as pl
from jax.experimental.pallas import tpu as pltpu
from jax.experimental.pallas import tpu_sc as plsc

N_TOK = 2048
D = 1024
N_EXP = 8
TOPK = 2
N_PAIR = N_TOK * TOPK

NW = 32
PAIRS_PER_TILE = N_PAIR // NW
TOK_PER_TILE = N_TOK // NW
HALF = TOK_PER_TILE // 2

BM = 256
M_TILES = N_PAIR // BM

_MESH = plsc.VectorSubcoreMesh(
    core_axis_name="c", subcore_axis_name="s", num_cores=2, num_subcores=16
)

_ONES16 = lambda: jnp.ones((16,), jnp.int32)
_IOTA16 = lambda: lax.iota(jnp.int32, 16)


@functools.partial(
    pl.kernel,
    compiler_params=pltpu.CompilerParams(needs_layout_passes=False),
    out_type=(
        jax.ShapeDtypeStruct((N_PAIR, D), jnp.float32),
        jax.ShapeDtypeStruct((N_TOK,), jnp.int32),
        jax.ShapeDtypeStruct((N_TOK,), jnp.int32),
        jax.ShapeDtypeStruct((16,), jnp.int32),
    ),
    mesh=_MESH,
    scratch_types=(
        pltpu.VMEM((N_PAIR,), jnp.int32),
        pltpu.VMEM((16,), jnp.int32),
        pltpu.VMEM((16,), jnp.int32),
        pltpu.VMEM((16,), jnp.int32),
        pltpu.VMEM((TOK_PER_TILE,), jnp.int32),
        pltpu.VMEM((TOK_PER_TILE,), jnp.int32),
        pltpu.VMEM((16,), jnp.int32),
        pltpu.VMEM((TOK_PER_TILE, D), jnp.float32),
        pltpu.SemaphoreType.DMA,
        pltpu.SemaphoreType.DMA,
    ),
)
def _routing(idx_hbm, x_hbm, xs_hbm, inv0_hbm, inv1_hbm, off_hbm,
             keys_v, hist_v, pre_v, cnt_v, inv0_v, inv1_v, off_v, xrows_v,
             sem0, sem1):
    wid = lax.axis_index("s") * 2 + lax.axis_index("c")
    zeros16 = jnp.zeros((16,), jnp.int32)

    pltpu.sync_copy(idx_hbm, keys_v)
    xcopy = pltpu.async_copy(x_hbm.at[pl.ds(wid * TOK_PER_TILE, TOK_PER_TILE)],
                             xrows_v, sem0)

    hist_v[...] = zeros16
    pre_v[...] = zeros16

    def _hist_step(i, _):
        k = keys_v[pl.ds(i * 16, 16)]
        plsc.addupdate_scatter(hist_v, [k], _ONES16())
        return 0

    lax.fori_loop(0, N_PAIR // 16, _hist_step, 0)

    def _pre_step(i, _):
        k = keys_v[pl.ds(i * 16, 16)]
        plsc.addupdate_scatter(pre_v, [k], _ONES16())
        return 0

    lax.fori_loop(0, wid * (PAIRS_PER_TILE // 16), _pre_step, 0)

    hist = hist_v[...]
    off = jnp.cumsum(hist) - hist
    cnt_v[...] = off + pre_v[...]

    for v in range(PAIRS_PER_TILE // 16):
        k = keys_v[pl.ds(wid * PAIRS_PER_TILE + v * 16, 16)]
        base = plsc.load_gather(cnt_v, [k])
        rank = zeros16
        for e in range(N_EXP):
            m = k == e
            cs = jnp.cumsum(m.astype(jnp.int32))
            rank = jnp.where(m, cs - 1, rank)
        pos = base + rank
        plsc.addupdate_scatter(cnt_v, [k], _ONES16())
        tok = (_IOTA16() + v * 16) // 2
        even = (_IOTA16() % 2) == 0
        plsc.store_scatter(inv0_v, [tok], pos, mask=even)
        plsc.store_scatter(inv1_v, [tok], pos, mask=jnp.logical_not(even))

    pltpu.sync_copy(inv0_v, inv0_hbm.at[pl.ds(wid * TOK_PER_TILE, TOK_PER_TILE)])
    pltpu.sync_copy(inv1_v, inv1_hbm.at[pl.ds(wid * TOK_PER_TILE, TOK_PER_TILE)])

    @pl.when(wid == 0)
    def _():
        off_v[...] = off
        pltpu.sync_copy(off_v, off_hbm)

    xcopy.wait()
    s0 = pltpu.async_copy(xrows_v, xs_hbm.at[inv0_v], sem1)
    s0.wait()
    s1 = pltpu.async_copy(xrows_v, xs_hbm.at[inv1_v], sem1)
    s1.wait()


def _gmm_body(off_ref, x_ref, w_ref, b_ref, y_ref):
    e = pl.program_id(0)
    m = pl.program_id(1)
    start = off_ref[e]
    end = off_ref[e + 1]
    row0 = m * BM

    @pl.when(jnp.logical_and(start < row0 + BM, end > row0))
    def _():
        rows = row0 + lax.broadcasted_iota(jnp.int32, (BM, 1), 0)
        mask = jnp.logical_and(rows >= start, rows < end)
        acc = jnp.dot(
            x_ref[...].astype(jnp.bfloat16),
            w_ref[0].astype(jnp.bfloat16),
            preferred_element_type=jnp.float32,
        )
        acc = acc + b_ref[0]
        y_ref[pl.ds(row0, BM), :] = jnp.where(
            mask, acc, y_ref[pl.ds(row0, BM), :]
        )


def _x_index(e, m, off_ref):
    lo = off_ref[e] // BM
    hi = jnp.maximum((off_ref[e + 1] - 1) // BM, lo)
    return (jnp.clip(m, lo, hi), 0)


def _gmm(off, xs, W, b3):
    grid_spec = pltpu.PrefetchScalarGridSpec(
        num_scalar_prefetch=1,
        grid=(N_EXP, M_TILES),
        in_specs=[
            pl.BlockSpec((BM, D), _x_index),
            pl.BlockSpec((1, D, D), lambda e, m, off_ref: (e, 0, 0)),
            pl.BlockSpec((1, 1, D), lambda e, m, off_ref: (e, 0, 0)),
        ],
        out_specs=pl.BlockSpec((N_PAIR, D), lambda e, m, off_ref: (0, 0)),
    )
    return pl.pallas_call(
        _gmm_body,
        grid_spec=grid_spec,
        out_shape=jax.ShapeDtypeStruct((N_PAIR, D), jnp.float32),
    )(off, xs, W, b3)


@functools.partial(
    pl.kernel,
    compiler_params=pltpu.CompilerParams(needs_layout_passes=False),
    out_type=jax.ShapeDtypeStruct((N_TOK, D), jnp.float32),
    mesh=_MESH,
    scratch_types=(
        pltpu.VMEM((HALF,), jnp.int32),
        pltpu.VMEM((HALF,), jnp.int32),
        pltpu.VMEM((TOK_PER_TILE,), jnp.float32),
        pltpu.VMEM((TOK_PER_TILE,), jnp.float32),
        pltpu.VMEM((HALF, D), jnp.float32),
        pltpu.VMEM((HALF, D), jnp.float32),
        pltpu.VMEM((HALF, D), jnp.float32),
        pltpu.SemaphoreType.DMA,
        pltpu.SemaphoreType.DMA,
    ),
)
def _combine(inv0_hbm, inv1_hbm, p0_hbm, p1_hbm, y_hbm, out_hbm,
             i0_v, i1_v, p0_v, p1_v, r0_v, r1_v, oc_v, sem0, sem1):
    wid = lax.axis_index("s") * 2 + lax.axis_index("c")
    base = wid * TOK_PER_TILE
    pltpu.sync_copy(p0_hbm.at[pl.ds(base, TOK_PER_TILE)], p0_v)
    pltpu.sync_copy(p1_hbm.at[pl.ds(base, TOK_PER_TILE)], p1_v)

    for h in range(2):
        hb = base + h * HALF
        pltpu.sync_copy(inv0_hbm.at[pl.ds(hb, HALF)], i0_v)
        pltpu.sync_copy(inv1_hbm.at[pl.ds(hb, HALF)], i1_v)
        g0 = pltpu.async_copy(y_hbm.at[i0_v], r0_v, sem0)
        g1 = pltpu.async_copy(y_hbm.at[i1_v], r1_v, sem1)
        g0.wait()
        g1.wait()

        def _fma(t, _):
            sel = jnp.full((16,), h * HALF + t, jnp.int32)
            s0 = plsc.load_gather(p0_v, [sel])
            s1 = plsc.load_gather(p1_v, [sel])
            for c in range(D // 16):
                oc_v[t, pl.ds(c * 16, 16)] = (
                    s0 * r0_v[t, pl.ds(c * 16, 16)]
                    + s1 * r1_v[t, pl.ds(c * 16, 16)]
                )
            return 0

        lax.fori_loop(0, HALF, _fma, 0)
        pltpu.sync_copy(oc_v, out_hbm.at[pl.ds(hb, HALF)])


def kernel(input_batch, probabilities, indices, W, b):
    idx32 = indices.reshape(N_PAIR).astype(jnp.int32)
    p0 = probabilities[:, 0]
    p1 = probabilities[:, 1]
    xs, inv0, inv1, off = _routing(idx32, input_batch)
    y = _gmm(off, xs, W, b.reshape(N_EXP, 1, D))
    out = _combine(inv0, inv1, p0, p1, y)
    total_loss = jnp.zeros((), dtype=jnp.float32)
    return (out, total_loss)


# speedup vs baseline: 1.1213x; 1.1213x over previous
import functools

import jax
import jax.numpy as jnp
from jax import lax
from jax.experimental import pallas as pl
from jax.experimental.pallas import tpu as pltpu
from jax.experimental.pallas import tpu_sc as plsc

N_TOK = 2048
D = 1024
N_EXP = 8
TOPK = 2
N_PAIR = N_TOK * TOPK

NW = 32
PAIRS_PER_TILE = N_PAIR // NW
TOK_PER_TILE = N_TOK // NW
HALF = TOK_PER_TILE // 2

BM = 256
M_TILES = N_PAIR // BM
S_PAD = 24

_MESH = plsc.VectorSubcoreMesh(
    core_axis_name="c", subcore_axis_name="s", num_cores=2, num_subcores=16
)

_ONES16 = lambda: jnp.ones((16,), jnp.int32)
_IOTA16 = lambda: lax.iota(jnp.int32, 16)


@functools.partial(
    pl.kernel,
    compiler_params=pltpu.CompilerParams(needs_layout_passes=False),
    out_type=(
        jax.ShapeDtypeStruct((N_PAIR, D), jnp.float32),
        jax.ShapeDtypeStruct((N_TOK,), jnp.int32),
        jax.ShapeDtypeStruct((N_TOK,), jnp.int32),
        jax.ShapeDtypeStruct((16,), jnp.int32),
        jax.ShapeDtypeStruct((S_PAD,), jnp.int32),
        jax.ShapeDtypeStruct((S_PAD,), jnp.int32),
    ),
    mesh=_MESH,
    scratch_types=(
        pltpu.VMEM((N_PAIR,), jnp.int32),
        pltpu.VMEM((16,), jnp.int32),
        pltpu.VMEM((16,), jnp.int32),
        pltpu.VMEM((16,), jnp.int32),
        pltpu.VMEM((TOK_PER_TILE,), jnp.int32),
        pltpu.VMEM((TOK_PER_TILE,), jnp.int32),
        pltpu.VMEM((16,), jnp.int32),
        pltpu.VMEM((S_PAD,), jnp.int32),
        pltpu.VMEM((S_PAD,), jnp.int32),
        pltpu.VMEM((TOK_PER_TILE, D), jnp.float32),
        pltpu.SemaphoreType.DMA,
        pltpu.SemaphoreType.DMA,
    ),
)
def _routing(idx_hbm, x_hbm, xs_hbm, inv0_hbm, inv1_hbm, off_hbm,
             me_hbm, ee_hbm,
             keys_v, hist_v, pre_v, cnt_v, inv0_v, inv1_v, off_v, me_v, ee_v,
             xrows_v, sem0, sem1):
    wid = lax.axis_index("s") * 2 + lax.axis_index("c")
    zeros16 = jnp.zeros((16,), jnp.int32)

    pltpu.sync_copy(idx_hbm, keys_v)
    xcopy = pltpu.async_copy(x_hbm.at[pl.ds(wid * TOK_PER_TILE, TOK_PER_TILE)],
                             xrows_v, sem0)

    hist_v[...] = zeros16
    pre_v[...] = zeros16

    def _hist_step(i, _):
        k = keys_v[pl.ds(i * 16, 16)]
        plsc.addupdate_scatter(hist_v, [k], _ONES16())
        return 0

    lax.fori_loop(0, N_PAIR // 16, _hist_step, 0)

    def _pre_step(i, _):
        k = keys_v[pl.ds(i * 16, 16)]
        plsc.addupdate_scatter(pre_v, [k], _ONES16())
        return 0

    lax.fori_loop(0, wid * (PAIRS_PER_TILE // 16), _pre_step, 0)

    hist = hist_v[...]
    off = jnp.cumsum(hist) - hist
    cnt_v[...] = off + pre_v[...]

    for v in range(PAIRS_PER_TILE // 16):
        k = keys_v[pl.ds(wid * PAIRS_PER_TILE + v * 16, 16)]
        base = plsc.load_gather(cnt_v, [k])
        rank = zeros16
        for e in range(N_EXP):
            m = k == e
            cs = jnp.cumsum(m.astype(jnp.int32))
            rank = jnp.where(m, cs - 1, rank)
        pos = base + rank
        plsc.addupdate_scatter(cnt_v, [k], _ONES16())
        tok = (_IOTA16() + v * 16) // 2
        even = (_IOTA16() % 2) == 0
        plsc.store_scatter(inv0_v, [tok], pos, mask=even)
        plsc.store_scatter(inv1_v, [tok], pos, mask=jnp.logical_not(even))

    pltpu.sync_copy(inv0_v, inv0_hbm.at[pl.ds(wid * TOK_PER_TILE, TOK_PER_TILE)])
    pltpu.sync_copy(inv1_v, inv1_hbm.at[pl.ds(wid * TOK_PER_TILE, TOK_PER_TILE)])

    @pl.when(wid == 0)
    def _():
        off_v[...] = off
        pltpu.sync_copy(off_v, off_hbm)
        lane0 = _IOTA16() == 0
        me_v[pl.ds(0, 16)] = jnp.full((16,), M_TILES - 1, jnp.int32)
        me_v[pl.ds(S_PAD - 16, 16)] = jnp.full((16,), M_TILES - 1, jnp.int32)
        ee_v[pl.ds(0, 16)] = jnp.full((16,), 14, jnp.int32)
        ee_v[pl.ds(S_PAD - 16, 16)] = jnp.full((16,), 14, jnp.int32)
        s = jnp.int32(0)
        for e in range(N_EXP):
            lane = _IOTA16()
            off_e = jnp.sum(jnp.where(lane == e, off, 0))
            off_e1 = jnp.sum(jnp.where(lane == e + 1, off, 0))
            lo = off_e // BM
            hi = jnp.maximum((off_e1 - 1) // BM, lo)

            def _emit(m, s_, _e=e):
                plsc.store_scatter(
                    me_v, [jnp.full((16,), s_, jnp.int32)],
                    jnp.full((16,), m, jnp.int32), mask=lane0)
                plsc.store_scatter(
                    ee_v, [jnp.full((16,), s_, jnp.int32)],
                    jnp.full((16,), _e, jnp.int32), mask=lane0)
                return s_ + 1

            s = lax.fori_loop(lo, hi + 1, _emit, s)
        pltpu.sync_copy(me_v, me_hbm)
        pltpu.sync_copy(ee_v, ee_hbm)

    xcopy.wait()
    s0 = pltpu.async_copy(xrows_v, xs_hbm.at[inv0_v], sem1)
    s0.wait()
    s1 = pltpu.async_copy(xrows_v, xs_hbm.at[inv1_v], sem1)
    s1.wait()


def _gmm_body(me_ref, ee_ref, off_ref, x_ref, w_ref, b_ref, y_ref):
    s = pl.program_id(0)
    e = ee_ref[s]
    m = me_ref[s]
    start = off_ref[e]
    end = off_ref[e + 1]
    row0 = m * BM

    @pl.when(jnp.logical_and(start < row0 + BM, end > row0))
    def _():
        rows = row0 + lax.broadcasted_iota(jnp.int32, (BM, 1), 0)
        mask = jnp.logical_and(rows >= start, rows < end)
        acc = jnp.dot(
            x_ref[...].astype(jnp.bfloat16),
            w_ref[0].astype(jnp.bfloat16),
            preferred_element_type=jnp.float32,
        )
        acc = acc + b_ref[0]
        y_ref[pl.ds(row0, BM), :] = jnp.where(
            mask, acc, y_ref[pl.ds(row0, BM), :]
        )


def _gmm(me, ee, off, xs, W, b3):
    grid_spec = pltpu.PrefetchScalarGridSpec(
        num_scalar_prefetch=3,
        grid=(S_PAD,),
        in_specs=[
            pl.BlockSpec((BM, D), lambda s, me_ref, ee_ref, off_ref: (me_ref[s], 0)),
            pl.BlockSpec(
                (1, D, D),
                lambda s, me_ref, ee_ref, off_ref: (
                    jnp.minimum(ee_ref[s], N_EXP - 1), 0, 0),
            ),
            pl.BlockSpec(
                (1, 1, D),
                lambda s, me_ref, ee_ref, off_ref: (
                    jnp.minimum(ee_ref[s], N_EXP - 1), 0, 0),
            ),
        ],
        out_specs=pl.BlockSpec((N_PAIR, D), lambda s, me_ref, ee_ref, off_ref: (0, 0)),
    )
    return pl.pallas_call(
        _gmm_body,
        grid_spec=grid_spec,
        out_shape=jax.ShapeDtypeStruct((N_PAIR, D), jnp.float32),
    )(me, ee, off, xs, W, b3)


@functools.partial(
    pl.kernel,
    compiler_params=pltpu.CompilerParams(needs_layout_passes=False),
    out_type=jax.ShapeDtypeStruct((N_TOK, D), jnp.float32),
    mesh=_MESH,
    scratch_types=(
        pltpu.VMEM((HALF,), jnp.int32),
        pltpu.VMEM((HALF,), jnp.int32),
        pltpu.VMEM((TOK_PER_TILE,), jnp.float32),
        pltpu.VMEM((TOK_PER_TILE,), jnp.float32),
        pltpu.VMEM((HALF, D), jnp.float32),
        pltpu.VMEM((HALF, D), jnp.float32),
        pltpu.VMEM((HALF, D), jnp.float32),
        pltpu.SemaphoreType.DMA,
        pltpu.SemaphoreType.DMA,
    ),
)
def _combine(inv0_hbm, inv1_hbm, p0_hbm, p1_hbm, y_hbm, out_hbm,
             i0_v, i1_v, p0_v, p1_v, r0_v, r1_v, oc_v, sem0, sem1):
    wid = lax.axis_index("s") * 2 + lax.axis_index("c")
    base = wid * TOK_PER_TILE
    pltpu.sync_copy(p0_hbm.at[pl.ds(base, TOK_PER_TILE)], p0_v)
    pltpu.sync_copy(p1_hbm.at[pl.ds(base, TOK_PER_TILE)], p1_v)

    for h in range(2):
        hb = base + h * HALF
        pltpu.sync_copy(inv0_hbm.at[pl.ds(hb, HALF)], i0_v)
        pltpu.sync_copy(inv1_hbm.at[pl.ds(hb, HALF)], i1_v)
        g0 = pltpu.async_copy(y_hbm.at[i0_v], r0_v, sem0)
        g1 = pltpu.async_copy(y_hbm.at[i1_v], r1_v, sem1)
        g0.wait()
        g1.wait()

        def _fma(t, _):
            sel = jnp.full((16,), h * HALF + t, jnp.int32)
            s0 = plsc.load_gather(p0_v, [sel])
            s1 = plsc.load_gather(p1_v, [sel])
            for c in range(D // 16):
                oc_v[t, pl.ds(c * 16, 16)] = (
                    s0 * r0_v[t, pl.ds(c * 16, 16)]
                    + s1 * r1_v[t, pl.ds(c * 16, 16)]
                )
            return 0

        lax.fori_loop(0, HALF, _fma, 0)
        pltpu.sync_copy(oc_v, out_hbm.at[pl.ds(hb, HALF)])


def kernel(input_batch, probabilities, indices, W, b):
    idx32 = indices.reshape(N_PAIR).astype(jnp.int32)
    p0 = probabilities[:, 0]
    p1 = probabilities[:, 1]
    xs, inv0, inv1, off, me, ee = _routing(idx32, input_batch)
    y = _gmm(me, ee, off, xs, W, b.reshape(N_EXP, 1, D))
    out = _combine(inv0, inv1, p0, p1, y)
    total_loss = jnp.zeros((), dtype=jnp.float32)
    return (out, total_loss)
